# baseline (speedup 1.0000x reference)
import functools

import jax
import jax.numpy as jnp
from jax import lax
from jax.experimental import pallas as pl
from jax.experimental.pallas import tpu as pltpu
from jax.experimental.pallas import tpu_sc as plsc

N = 10000
E = 320000
D = 128

NC = 2
NS = 16
NW = NC * NS

K = 120
EPW = E // NW
CHUNKS = EPW // K
KT = EPW - CHUNKS * K

N_ACC = 10112
ZR = N_ACC // NS
OUTR = 632
OUTR_LAST = N - 15 * OUTR

R_BLK = 2000
GRID = N // R_BLK

_MESH = plsc.VectorSubcoreMesh(core_axis_name="c", subcore_axis_name="s")


def _writeback(sid, cid, acc, out_hbm):
    r0 = sid * OUTR

    @pl.when(sid < NS - 1)
    def _():
        pltpu.sync_copy(acc.at[pl.ds(r0, OUTR)],
                        out_hbm.at[cid, pl.ds(r0, OUTR)])

    @pl.when(sid == NS - 1)
    def _():
        r1 = (NS - 1) * OUTR
        pltpu.sync_copy(acc.at[pl.ds(r1, OUTR_LAST)],
                        out_hbm.at[cid, pl.ds(r1, OUTR_LAST)])


@functools.partial(
    pl.kernel,
    out_type=jax.ShapeDtypeStruct((NC, N, D), jnp.float32),
    mesh=_MESH,
    scratch_types=[
        pltpu.VMEM_SHARED((N_ACC, D), jnp.float32),
        pltpu.VMEM((K,), jnp.int32),
        pltpu.VMEM((K,), jnp.int32),
        pltpu.VMEM((KT,), jnp.int32),
        pltpu.VMEM((K, D), jnp.float32),
        pltpu.SemaphoreType.DMA,
        pltpu.SemaphoreType.DMA,
    ],
)
def _sc_degree(dst_hbm, zeros_hbm, ones_hbm, deg_out, deg_acc,
               didx0, didx1, didx_t, ones, isem0, isem1):
    cid = lax.axis_index("c")
    sid = lax.axis_index("s")
    wid = sid * NC + cid

    pltpu.sync_copy(ones_hbm, ones)
    pltpu.sync_copy(zeros_hbm, deg_acc.at[pl.ds(sid * ZR, ZR)])
    plsc.subcore_barrier()

    base0 = wid * EPW
    dbufs = ((didx0, isem0), (didx1, isem1))

    def edge_slice(k):
        return dst_hbm.at[pl.ds(pl.multiple_of(base0 + k * K, 8), K)]

    pltpu.async_copy(edge_slice(0), didx0, isem0)
    pltpu.async_copy(edge_slice(1), didx1, isem1)

    def pair(g, carry):
        for b in range(2):
            k = 2 * g + b
            didx, isem = dbufs[b]

            @pl.when(k < CHUNKS)
            def _():
                pltpu.make_async_copy(edge_slice(k), didx, isem).wait()
                pltpu.sync_copy(ones, deg_acc.at[didx], add=True)

                @pl.when(k + 2 < CHUNKS)
                def _():
                    pltpu.async_copy(edge_slice(k + 2), didx, isem)
        return carry

    lax.fori_loop(0, (CHUNKS + 1) // 2, pair, 0)
    pltpu.sync_copy(dst_hbm.at[pl.ds(pl.multiple_of(base0 + CHUNKS * K, 8), KT)],
                    didx_t)
    pltpu.sync_copy(ones.at[pl.ds(0, KT)], deg_acc.at[didx_t], add=True)
    plsc.subcore_barrier()
    _writeback(sid, cid, deg_acc, deg_out)


@functools.partial(
    pl.kernel,
    out_type=jax.ShapeDtypeStruct((NC, N, D), jnp.float32),
    mesh=_MESH,
    scratch_types=(
        [pltpu.VMEM_SHARED((N_ACC, D), jnp.float32)]
        + [pltpu.VMEM((K,), jnp.int32)] * 6
        + [pltpu.VMEM((KT,), jnp.int32)] * 2
        + [pltpu.VMEM((K, D), jnp.float32)] * 3
        + [pltpu.SemaphoreType.DMA] * 6
    ),
)
def _sc_scatter(x_hbm, src_hbm, dst_hbm, zeros_hbm, part_out, acc,
                sidx0, sidx1, sidx2, didx0, didx1, didx2, sidx_t, didx_t,
                rows0, rows1, rows2,
                isem0, isem1, isem2, gsem0, gsem1, gsem2):
    cid = lax.axis_index("c")
    sid = lax.axis_index("s")
    wid = sid * NC + cid

    pltpu.sync_copy(zeros_hbm, acc.at[pl.ds(sid * ZR, ZR)])
    plsc.subcore_barrier()

    base0 = wid * EPW
    bufs = ((sidx0, didx0, rows0, isem0, gsem0),
            (sidx1, didx1, rows1, isem1, gsem1),
            (sidx2, didx2, rows2, isem2, gsem2))
    NB = 3

    def src_slice(k):
        return src_hbm.at[pl.ds(pl.multiple_of(base0 + k * K, 8), K)]

    def dst_slice(k):
        return dst_hbm.at[pl.ds(pl.multiple_of(base0 + k * K, 8), K)]

    def issue_idx(k, b):
        sidx, didx, _, isem, _ = bufs[b]
        pltpu.async_copy(src_slice(k), sidx, isem)
        pltpu.async_copy(dst_slice(k), didx, isem)

    def wait_idx(k, b):
        sidx, didx, _, isem, _ = bufs[b]
        pltpu.make_async_copy(src_slice(k), sidx, isem).wait()
        pltpu.make_async_copy(dst_slice(k), didx, isem).wait()

    def issue_gather(b):
        sidx, _, rows, _, gsem = bufs[b]
        pltpu.async_copy(x_hbm.at[sidx], rows, gsem)

    def wait_gather(b):
        sidx, _, rows, _, gsem = bufs[b]
        pltpu.make_async_copy(x_hbm.at[sidx], rows, gsem).wait()

    def scatter(b):
        _, didx, rows, _, _ = bufs[b]
        pltpu.sync_copy(rows, acc.at[didx], add=True)

    for b in range(NB):
        issue_idx(b, b)
    for b in range(NB - 1):
        wait_idx(b, b)
        issue_gather(b)

    def triple(g, carry):
        for b in range(NB):
            k = NB * g + b

            @pl.when(k < CHUNKS)
            def _():
                wait_gather(b)
                nb = (b + NB - 1) % NB

                @pl.when(k + NB - 1 < CHUNKS)
                def _():
                    wait_idx(k + NB - 1, nb)
                    issue_gather(nb)

                scatter(b)

                @pl.when(k + NB < CHUNKS)
                def _():
                    issue_idx(k + NB, b)
        return carry

    lax.fori_loop(0, (CHUNKS + NB - 1) // NB, triple, 0)
    tb = pl.multiple_of(base0 + CHUNKS * K, 8)
    pltpu.sync_copy(src_hbm.at[pl.ds(tb, KT)], sidx_t)
    pltpu.sync_copy(dst_hbm.at[pl.ds(tb, KT)], didx_t)
    pltpu.async_copy(x_hbm.at[sidx_t], rows0.at[pl.ds(0, KT)], gsem0).wait()
    pltpu.sync_copy(rows0.at[pl.ds(0, KT)], acc.at[didx_t], add=True)
    plsc.subcore_barrier()
    _writeback(sid, cid, acc, part_out)


def _tc_scale_body(x_ref, deg_ref, out_ref):
    d = deg_ref[0, :, 0:1] + deg_ref[1, :, 0:1]
    dis = lax.rsqrt(jnp.maximum(d, 1.0))
    out_ref[...] = x_ref[...] * dis


def _tc_scale(x, deg_parts):
    return pl.pallas_call(
        _tc_scale_body,
        grid=(GRID,),
        in_specs=[
            pl.BlockSpec((R_BLK, D), lambda i: (i, 0)),
            pl.BlockSpec((NC, R_BLK, D), lambda i: (0, i, 0)),
        ],
        out_specs=pl.BlockSpec((R_BLK, D), lambda i: (i, 0)),
        out_shape=jax.ShapeDtypeStruct((N, D), jnp.float32),
    )(x, deg_parts)


def _tc_final_body(scal_ref, part_ref, deg_ref, h0_ref, w_ref, out_ref):
    theta = scal_ref[0, 0]
    alpha = scal_ref[0, 1]
    d = deg_ref[0, :, 0:1] + deg_ref[1, :, 0:1]
    dis = lax.rsqrt(jnp.maximum(d, 1.0))
    h_acc = part_ref[0] + part_ref[1]
    support = (1.0 - alpha) * (h_acc * dis) + alpha * h0_ref[...]
    mm = jnp.dot(support, w_ref[...], preferred_element_type=jnp.float32)
    out_ref[...] = theta * mm + (1.0 - theta) * support


def _tc_final(part, deg_parts, h0, W, scal):
    return pl.pallas_call(
        _tc_final_body,
        grid=(GRID,),
        in_specs=[
            pl.BlockSpec(memory_space=pltpu.SMEM),
            pl.BlockSpec((NC, R_BLK, D), lambda i: (0, i, 0)),
            pl.BlockSpec((NC, R_BLK, D), lambda i: (0, i, 0)),
            pl.BlockSpec((R_BLK, D), lambda i: (i, 0)),
            pl.BlockSpec((D, D), lambda i: (0, 0)),
        ],
        out_specs=pl.BlockSpec((R_BLK, D), lambda i: (i, 0)),
        out_shape=jax.ShapeDtypeStruct((N, D), jnp.float32),
    )(scal, part, deg_parts, h0, W)


def kernel(inputs, edge_index, h0, lamda, alpha, l, W):
    theta = jnp.log(lamda / l + 1)
    scal = jnp.reshape(
        jnp.stack([theta, alpha]).astype(jnp.float32), (1, 2))

    src = edge_index[0]
    dst = edge_index[1]

    zstripe = jnp.zeros((ZR, D), jnp.float32)
    ones128 = jnp.ones((K, D), jnp.float32)

    deg_parts = _sc_degree(dst, zstripe, ones128)
    x_scaled = _tc_scale(inputs, deg_parts)
    part = _sc_scatter(x_scaled, src, dst, zstripe)
    return _tc_final(part, deg_parts, h0, W, scal)


# speedup vs baseline: 1.2600x; 1.2600x over previous
import functools

import jax
import jax.numpy as jnp
from jax import lax
from jax.experimental import pallas as pl
from jax.experimental.pallas import tpu as pltpu
from jax.experimental.pallas import tpu_sc as plsc

N = 10000
E = 320000
D = 128

NC = 2
NS = 16
NW = NC * NS

K = 120
EPW = E // NW
CHUNKS = EPW // K
KT = EPW - CHUNKS * K

N_ACC = 10112
ZR = N_ACC // NS
OUTR = 632
OUTR_LAST = N - 15 * OUTR

R_BLK = 2000
GRID = N // R_BLK

_MESH = plsc.VectorSubcoreMesh(core_axis_name="c", subcore_axis_name="s")


def _writeback(sid, cid, acc, out_hbm):
    r0 = sid * OUTR

    @pl.when(sid < NS - 1)
    def _():
        pltpu.sync_copy(acc.at[pl.ds(r0, OUTR)],
                        out_hbm.at[cid, pl.ds(r0, OUTR)])

    @pl.when(sid == NS - 1)
    def _():
        r1 = (NS - 1) * OUTR
        pltpu.sync_copy(acc.at[pl.ds(r1, OUTR_LAST)],
                        out_hbm.at[cid, pl.ds(r1, OUTR_LAST)])


RED = 640
RED_LAST = N_ACC - 15 * RED
VSTEPS = EPW // 16


@functools.partial(
    pl.kernel,
    out_type=jax.ShapeDtypeStruct((NC * N_ACC,), jnp.float32),
    mesh=_MESH,
    scratch_types=[
        pltpu.VMEM_SHARED((NS * N_ACC,), jnp.float32),
        pltpu.VMEM((EPW,), jnp.int32),
        pltpu.VMEM((N_ACC,), jnp.float32),
        pltpu.VMEM((16 * RED,), jnp.float32),
        pltpu.SemaphoreType.DMA,
    ],
    compiler_params=pltpu.CompilerParams(needs_layout_passes=False),
)
def _sc_degree(dst_hbm, zeros_hbm, deg_out, hist_sh, idxv, hist, vbuf, rsem):
    cid = lax.axis_index("c")
    sid = lax.axis_index("s")
    wid = sid * NC + cid

    pltpu.sync_copy(zeros_hbm, hist)
    pltpu.sync_copy(dst_hbm.at[pl.ds(pl.multiple_of(wid * EPW, 8), EPW)], idxv)

    def step(j, carry):
        raw = idxv[pl.ds(j * 16, 16)]
        c_fwd, _ = plsc.scan_count(raw)
        cr, _ = plsc.scan_count(lax.rev(raw, (0,)))
        c_rev = lax.rev(cr, (0,))
        plsc.addupdate_scatter(hist, [raw], c_fwd.astype(jnp.float32),
                               mask=(c_rev == 1))
        return carry

    lax.fori_loop(0, VSTEPS, step, 0)

    pltpu.sync_copy(hist, hist_sh.at[pl.ds(sid * N_ACC, N_ACC)])
    plsc.subcore_barrier()

    def reduce_and_write(red, r0):
        for h in range(NS):
            pltpu.async_copy(hist_sh.at[pl.ds(h * N_ACC + r0, red)],
                             vbuf.at[pl.ds(h * red, red)], rsem)
        for h in range(NS):
            pltpu.make_async_copy(hist_sh.at[pl.ds(h * N_ACC + r0, red)],
                                  vbuf.at[pl.ds(h * red, red)], rsem).wait()

        def vsum(v, carry):
            t = vbuf[pl.ds(v * 16, 16)]
            for h in range(1, NS):
                t = t + vbuf[pl.ds(h * red + v * 16, 16)]
            hist[pl.ds(v * 16, 16)] = t
            return carry

        lax.fori_loop(0, red // 16, vsum, 0)
        pltpu.sync_copy(hist.at[pl.ds(0, red)],
                        deg_out.at[pl.ds(cid * N_ACC + r0, red)])

    @pl.when(sid < NS - 1)
    def _():
        reduce_and_write(RED, sid * RED)

    @pl.when(sid == NS - 1)
    def _():
        reduce_and_write(RED_LAST, (NS - 1) * RED)


@functools.partial(
    pl.kernel,
    out_type=jax.ShapeDtypeStruct((NC, N, D), jnp.float32),
    mesh=_MESH,
    scratch_types=(
        [pltpu.VMEM_SHARED((N_ACC, D), jnp.float32)]
        + [pltpu.VMEM((K,), jnp.int32)] * 6
        + [pltpu.VMEM((KT,), jnp.int32)] * 2
        + [pltpu.VMEM((K, D), jnp.float32)] * 3
        + [pltpu.SemaphoreType.DMA] * 6
    ),
)
def _sc_scatter(x_hbm, src_hbm, dst_hbm, zeros_hbm, part_out, acc,
                sidx0, sidx1, sidx2, didx0, didx1, didx2, sidx_t, didx_t,
                rows0, rows1, rows2,
                isem0, isem1, isem2, gsem0, gsem1, gsem2):
    cid = lax.axis_index("c")
    sid = lax.axis_index("s")
    wid = sid * NC + cid

    pltpu.sync_copy(zeros_hbm, acc.at[pl.ds(sid * ZR, ZR)])
    plsc.subcore_barrier()

    base0 = wid * EPW
    bufs = ((sidx0, didx0, rows0, isem0, gsem0),
            (sidx1, didx1, rows1, isem1, gsem1),
            (sidx2, didx2, rows2, isem2, gsem2))
    NB = 3

    def src_slice(k):
        return src_hbm.at[pl.ds(pl.multiple_of(base0 + k * K, 8), K)]

    def dst_slice(k):
        return dst_hbm.at[pl.ds(pl.multiple_of(base0 + k * K, 8), K)]

    def issue_idx(k, b):
        sidx, didx, _, isem, _ = bufs[b]
        pltpu.async_copy(src_slice(k), sidx, isem)
        pltpu.async_copy(dst_slice(k), didx, isem)

    def wait_idx(k, b):
        sidx, didx, _, isem, _ = bufs[b]
        pltpu.make_async_copy(src_slice(k), sidx, isem).wait()
        pltpu.make_async_copy(dst_slice(k), didx, isem).wait()

    def issue_gather(b):
        sidx, _, rows, _, gsem = bufs[b]
        pltpu.async_copy(x_hbm.at[sidx], rows, gsem)

    def wait_gather(b):
        sidx, _, rows, _, gsem = bufs[b]
        pltpu.make_async_copy(x_hbm.at[sidx], rows, gsem).wait()

    def scatter(b):
        _, didx, rows, _, _ = bufs[b]
        pltpu.sync_copy(rows, acc.at[didx], add=True)

    for b in range(NB):
        issue_idx(b, b)
    for b in range(NB - 1):
        wait_idx(b, b)
        issue_gather(b)

    def triple(g, carry):
        for b in range(NB):
            k = NB * g + b

            @pl.when(k < CHUNKS)
            def _():
                wait_gather(b)
                nb = (b + NB - 1) % NB

                @pl.when(k + NB - 1 < CHUNKS)
                def _():
                    wait_idx(k + NB - 1, nb)
                    issue_gather(nb)

                scatter(b)

                @pl.when(k + NB < CHUNKS)
                def _():
                    issue_idx(k + NB, b)
        return carry

    lax.fori_loop(0, (CHUNKS + NB - 1) // NB, triple, 0)
    tb = pl.multiple_of(base0 + CHUNKS * K, 8)
    pltpu.sync_copy(src_hbm.at[pl.ds(tb, KT)], sidx_t)
    pltpu.sync_copy(dst_hbm.at[pl.ds(tb, KT)], didx_t)
    pltpu.async_copy(x_hbm.at[sidx_t], rows0.at[pl.ds(0, KT)], gsem0).wait()
    pltpu.sync_copy(rows0.at[pl.ds(0, KT)], acc.at[didx_t], add=True)
    plsc.subcore_barrier()
    _writeback(sid, cid, acc, part_out)


def _tc_scale_body(x_ref, deg_ref, out_ref):
    d = deg_ref[:, 0:1] + deg_ref[:, 1:2]
    dis = lax.rsqrt(jnp.maximum(d, 1.0))
    out_ref[...] = x_ref[...] * dis


def _tc_scale(x, deg_cols):
    return pl.pallas_call(
        _tc_scale_body,
        grid=(GRID,),
        in_specs=[
            pl.BlockSpec((R_BLK, D), lambda i: (i, 0)),
            pl.BlockSpec((R_BLK, NC), lambda i: (i, 0)),
        ],
        out_specs=pl.BlockSpec((R_BLK, D), lambda i: (i, 0)),
        out_shape=jax.ShapeDtypeStruct((N, D), jnp.float32),
    )(x, deg_cols)


def _tc_final_body(scal_ref, part_ref, deg_ref, h0_ref, w_ref, out_ref):
    theta = scal_ref[0, 0]
    alpha = scal_ref[0, 1]
    d = deg_ref[:, 0:1] + deg_ref[:, 1:2]
    dis = lax.rsqrt(jnp.maximum(d, 1.0))
    h_acc = part_ref[0] + part_ref[1]
    support = (1.0 - alpha) * (h_acc * dis) + alpha * h0_ref[...]
    mm = jnp.dot(support, w_ref[...], preferred_element_type=jnp.float32)
    out_ref[...] = theta * mm + (1.0 - theta) * support


def _tc_final(part, deg_cols, h0, W, scal):
    return pl.pallas_call(
        _tc_final_body,
        grid=(GRID,),
        in_specs=[
            pl.BlockSpec(memory_space=pltpu.SMEM),
            pl.BlockSpec((NC, R_BLK, D), lambda i: (0, i, 0)),
            pl.BlockSpec((R_BLK, NC), lambda i: (i, 0)),
            pl.BlockSpec((R_BLK, D), lambda i: (i, 0)),
            pl.BlockSpec((D, D), lambda i: (0, 0)),
        ],
        out_specs=pl.BlockSpec((R_BLK, D), lambda i: (i, 0)),
        out_shape=jax.ShapeDtypeStruct((N, D), jnp.float32),
    )(scal, part, deg_cols, h0, W)


def kernel(inputs, edge_index, h0, lamda, alpha, l, W):
    theta = jnp.log(lamda / l + 1)
    scal = jnp.reshape(
        jnp.stack([theta, alpha]).astype(jnp.float32), (1, 2))

    src = edge_index[0]
    dst = edge_index[1]

    zstripe = jnp.zeros((ZR, D), jnp.float32)
    zeros1 = jnp.zeros((N_ACC,), jnp.float32)

    deg_flat = _sc_degree(dst, zeros1)
    deg_cols = jnp.transpose(jnp.reshape(deg_flat, (NC, N_ACC)))[:N]
    x_scaled = _tc_scale(inputs, deg_cols)
    part = _sc_scatter(x_scaled, src, dst, zstripe)
    return _tc_final(part, deg_cols, h0, W, scal)


# speedup vs baseline: 1.4367x; 1.1403x over previous
import functools

import jax
import jax.numpy as jnp
from jax import lax
from jax.experimental import pallas as pl
from jax.experimental.pallas import tpu as pltpu
from jax.experimental.pallas import tpu_sc as plsc

N = 10000
E = 320000
D = 128

NC = 2
NS = 16
NW = NC * NS

K = 120
EPW = E // NW
CHUNKS = EPW // K
KT = EPW - CHUNKS * K

N_ACC = 10112
ZR = N_ACC // NS
OUTR = 632
OUTR_LAST = N - 15 * OUTR

R_BLK = 2000
GRID = N // R_BLK

_MESH = plsc.VectorSubcoreMesh(core_axis_name="c", subcore_axis_name="s")


def _writeback(sid, cid, acc, out_hbm):
    r0 = sid * OUTR

    @pl.when(sid < NS - 1)
    def _():
        pltpu.sync_copy(acc.at[pl.ds(r0, OUTR)],
                        out_hbm.at[cid, pl.ds(r0, OUTR)])

    @pl.when(sid == NS - 1)
    def _():
        r1 = (NS - 1) * OUTR
        pltpu.sync_copy(acc.at[pl.ds(r1, OUTR_LAST)],
                        out_hbm.at[cid, pl.ds(r1, OUTR_LAST)])


RED = 640
RED_LAST = N_ACC - 15 * RED
VSTEPS = EPW // 16


@functools.partial(
    pl.kernel,
    out_type=jax.ShapeDtypeStruct((NC * N_ACC,), jnp.float32),
    mesh=_MESH,
    scratch_types=[
        pltpu.VMEM_SHARED((NS * N_ACC,), jnp.float32),
        pltpu.VMEM((EPW,), jnp.int32),
        pltpu.VMEM((N_ACC,), jnp.float32),
        pltpu.VMEM((16 * RED,), jnp.float32),
        pltpu.SemaphoreType.DMA,
    ],
    compiler_params=pltpu.CompilerParams(needs_layout_passes=False),
)
def _sc_degree(dst_hbm, zeros_hbm, deg_out, hist_sh, idxv, hist, vbuf, rsem):
    cid = lax.axis_index("c")
    sid = lax.axis_index("s")
    wid = sid * NC + cid

    pltpu.sync_copy(zeros_hbm, hist)
    pltpu.sync_copy(dst_hbm.at[pl.ds(pl.multiple_of(wid * EPW, 8), EPW)], idxv)

    def step(j, carry):
        raw = idxv[pl.ds(j * 16, 16)]
        c_fwd, _ = plsc.scan_count(raw)
        cr, _ = plsc.scan_count(lax.rev(raw, (0,)))
        c_rev = lax.rev(cr, (0,))
        plsc.addupdate_scatter(hist, [raw], c_fwd.astype(jnp.float32),
                               mask=(c_rev == 1))
        return carry

    lax.fori_loop(0, VSTEPS, step, 0)

    pltpu.sync_copy(hist, hist_sh.at[pl.ds(sid * N_ACC, N_ACC)])
    plsc.subcore_barrier()

    def reduce_and_write(red, r0):
        for h in range(NS):
            pltpu.async_copy(hist_sh.at[pl.ds(h * N_ACC + r0, red)],
                             vbuf.at[pl.ds(h * red, red)], rsem)
        for h in range(NS):
            pltpu.make_async_copy(hist_sh.at[pl.ds(h * N_ACC + r0, red)],
                                  vbuf.at[pl.ds(h * red, red)], rsem).wait()

        def vsum(v, carry):
            t = vbuf[pl.ds(v * 16, 16)]
            for h in range(1, NS):
                t = t + vbuf[pl.ds(h * red + v * 16, 16)]
            hist[pl.ds(v * 16, 16)] = t
            return carry

        lax.fori_loop(0, red // 16, vsum, 0)
        pltpu.sync_copy(hist.at[pl.ds(0, red)],
                        deg_out.at[pl.ds(cid * N_ACC + r0, red)])

    @pl.when(sid < NS - 1)
    def _():
        reduce_and_write(RED, sid * RED)

    @pl.when(sid == NS - 1)
    def _():
        reduce_and_write(RED_LAST, (NS - 1) * RED)


@functools.partial(
    pl.kernel,
    out_type=jax.ShapeDtypeStruct((NC, N, D), jnp.float32),
    mesh=_MESH,
    scratch_types=(
        [pltpu.VMEM_SHARED((N_ACC, D), jnp.float32)]
        + [pltpu.VMEM((K,), jnp.int32)] * 6
        + [pltpu.VMEM((KT,), jnp.int32)] * 2
        + [pltpu.VMEM((K, D), jnp.float32)] * 3
        + [pltpu.SemaphoreType.DMA] * 12
    ),
)
def _sc_scatter(x_hbm, src_hbm, dst_hbm, zeros_hbm, part_out, acc,
                sidx0, sidx1, sidx2, didx0, didx1, didx2, sidx_t, didx_t,
                rows0, rows1, rows2,
                isem0, isem1, isem2, gsem0, gsem1, gsem2,
                ssem0, ssem1, ssem2, dsem0, dsem1, dsem2):
    cid = lax.axis_index("c")
    sid = lax.axis_index("s")
    wid = sid * NC + cid

    pltpu.sync_copy(zeros_hbm, acc.at[pl.ds(sid * ZR, ZR)])
    plsc.subcore_barrier()

    base0 = wid * EPW
    bufs = ((sidx0, didx0, rows0, isem0, gsem0, ssem0, dsem0),
            (sidx1, didx1, rows1, isem1, gsem1, ssem1, dsem1),
            (sidx2, didx2, rows2, isem2, gsem2, ssem2, dsem2))
    NB = 3

    def src_slice(k):
        return src_hbm.at[pl.ds(pl.multiple_of(base0 + k * K, 8), K)]

    def dst_slice(k):
        return dst_hbm.at[pl.ds(pl.multiple_of(base0 + k * K, 8), K)]

    def issue_sidx(k, b):
        sidx, _, _, isem, _, _, _ = bufs[b]
        pltpu.async_copy(src_slice(k), sidx, isem)

    def wait_sidx(k, b):
        sidx, _, _, isem, _, _, _ = bufs[b]
        pltpu.make_async_copy(src_slice(k), sidx, isem).wait()

    def issue_didx(k, b):
        _, didx, _, _, _, _, dsem = bufs[b]
        pltpu.async_copy(dst_slice(k), didx, dsem)

    def wait_didx(k, b):
        _, didx, _, _, _, _, dsem = bufs[b]
        pltpu.make_async_copy(dst_slice(k), didx, dsem).wait()

    def issue_gather(b):
        sidx, _, rows, _, gsem, _, _ = bufs[b]
        pltpu.async_copy(x_hbm.at[sidx], rows, gsem)

    def wait_gather(b):
        sidx, _, rows, _, gsem, _, _ = bufs[b]
        pltpu.make_async_copy(x_hbm.at[sidx], rows, gsem).wait()

    def issue_scatter(b):
        _, didx, rows, _, _, ssem, _ = bufs[b]
        pltpu.async_copy(rows, acc.at[didx], ssem, add=True)

    def wait_scatter(b):
        _, didx, rows, _, _, ssem, _ = bufs[b]
        pltpu.make_async_copy(rows, acc.at[didx], ssem).wait()

    for b in range(NB):
        issue_sidx(b, b)
    issue_didx(0, 0)
    issue_didx(1, 1)
    issue_didx(2, 2)
    for b in range(NB - 1):
        wait_sidx(b, b)
        issue_gather(b)

    def triple(g, carry):
        for b in range(NB):
            k = NB * g + b

            @pl.when(k < CHUNKS)
            def _():
                wait_gather(b)
                nb = (b + NB - 1) % NB

                @pl.when(k + NB < CHUNKS)
                def _():
                    issue_sidx(k + NB, b)

                @pl.when(k + NB - 1 < CHUNKS)
                def _():
                    @pl.when(k >= 1)
                    def _():
                        wait_scatter(nb)
                        issue_didx(k + NB - 1, nb)
                    wait_sidx(k + NB - 1, nb)
                    issue_gather(nb)

                wait_didx(k, b)
                issue_scatter(b)
        return carry

    lax.fori_loop(0, (CHUNKS + NB - 1) // NB, triple, 0)
    for j in range(CHUNKS - NB, CHUNKS):
        wait_scatter(j % NB)
    tb = pl.multiple_of(base0 + CHUNKS * K, 8)
    pltpu.sync_copy(src_hbm.at[pl.ds(tb, KT)], sidx_t)
    pltpu.sync_copy(dst_hbm.at[pl.ds(tb, KT)], didx_t)
    pltpu.async_copy(x_hbm.at[sidx_t], rows0.at[pl.ds(0, KT)], gsem0).wait()
    pltpu.sync_copy(rows0.at[pl.ds(0, KT)], acc.at[didx_t], add=True)
    plsc.subcore_barrier()
    _writeback(sid, cid, acc, part_out)


def _tc_scale_body(x_ref, deg_ref, out_ref):
    d = deg_ref[:, 0:1] + deg_ref[:, 1:2]
    dis = lax.rsqrt(jnp.maximum(d, 1.0))
    out_ref[...] = x_ref[...] * dis


def _tc_scale(x, deg_cols):
    return pl.pallas_call(
        _tc_scale_body,
        grid=(GRID,),
        in_specs=[
            pl.BlockSpec((R_BLK, D), lambda i: (i, 0)),
            pl.BlockSpec((R_BLK, NC), lambda i: (i, 0)),
        ],
        out_specs=pl.BlockSpec((R_BLK, D), lambda i: (i, 0)),
        out_shape=jax.ShapeDtypeStruct((N, D), jnp.float32),
    )(x, deg_cols)


def _tc_final_body(scal_ref, part_ref, deg_ref, h0_ref, w_ref, out_ref):
    theta = scal_ref[0, 0]
    alpha = scal_ref[0, 1]
    d = deg_ref[:, 0:1] + deg_ref[:, 1:2]
    dis = lax.rsqrt(jnp.maximum(d, 1.0))
    h_acc = part_ref[0] + part_ref[1]
    support = (1.0 - alpha) * (h_acc * dis) + alpha * h0_ref[...]
    mm = jnp.dot(support, w_ref[...], preferred_element_type=jnp.float32)
    out_ref[...] = theta * mm + (1.0 - theta) * support


def _tc_final(part, deg_cols, h0, W, scal):
    return pl.pallas_call(
        _tc_final_body,
        grid=(GRID,),
        in_specs=[
            pl.BlockSpec(memory_space=pltpu.SMEM),
            pl.BlockSpec((NC, R_BLK, D), lambda i: (0, i, 0)),
            pl.BlockSpec((R_BLK, NC), lambda i: (i, 0)),
            pl.BlockSpec((R_BLK, D), lambda i: (i, 0)),
            pl.BlockSpec((D, D), lambda i: (0, 0)),
        ],
        out_specs=pl.BlockSpec((R_BLK, D), lambda i: (i, 0)),
        out_shape=jax.ShapeDtypeStruct((N, D), jnp.float32),
    )(scal, part, deg_cols, h0, W)


def kernel(inputs, edge_index, h0, lamda, alpha, l, W):
    theta = jnp.log(lamda / l + 1)
    scal = jnp.reshape(
        jnp.stack([theta, alpha]).astype(jnp.float32), (1, 2))

    src = edge_index[0]
    dst = edge_index[1]

    zstripe = jnp.zeros((ZR, D), jnp.float32)
    zeros1 = jnp.zeros((N_ACC,), jnp.float32)

    deg_flat = _sc_degree(dst, zeros1)
    deg_cols = jnp.transpose(jnp.reshape(deg_flat, (NC, N_ACC)))[:N]
    x_scaled = _tc_scale(inputs, deg_cols)
    part = _sc_scatter(x_scaled, src, dst, zstripe)
    return _tc_final(part, deg_cols, h0, W, scal)


# speedup vs baseline: 1.4511x; 1.0100x over previous
import functools

import jax
import jax.numpy as jnp
from jax import lax
from jax.experimental import pallas as pl
from jax.experimental.pallas import tpu as pltpu
from jax.experimental.pallas import tpu_sc as plsc

N = 10000
E = 320000
D = 128

NC = 2
NS = 16
NW = NC * NS

K = 96
EPW = E // NW
CHUNKS = EPW // K
KT = EPW - CHUNKS * K

N_ACC = 10112
N_FEAT = 10040
ZR = 632
ZR_LAST = N_FEAT - 15 * ZR
OUTR = 632
OUTR_LAST = N - 15 * OUTR

R_BLK = 2000
GRID = N // R_BLK

_MESH = plsc.VectorSubcoreMesh(core_axis_name="c", subcore_axis_name="s")


def _writeback(sid, cid, acc, out_hbm):
    r0 = sid * OUTR

    @pl.when(sid < NS - 1)
    def _():
        pltpu.sync_copy(acc.at[pl.ds(r0, OUTR)],
                        out_hbm.at[cid, pl.ds(r0, OUTR)])

    @pl.when(sid == NS - 1)
    def _():
        r1 = (NS - 1) * OUTR
        pltpu.sync_copy(acc.at[pl.ds(r1, OUTR_LAST)],
                        out_hbm.at[cid, pl.ds(r1, OUTR_LAST)])


RED = 640
RED_LAST = N_ACC - 15 * RED
VSTEPS = EPW // 16


@functools.partial(
    pl.kernel,
    out_type=jax.ShapeDtypeStruct((NC * N_ACC,), jnp.float32),
    mesh=_MESH,
    scratch_types=[
        pltpu.VMEM_SHARED((NS * N_ACC,), jnp.float32),
        pltpu.VMEM((EPW,), jnp.int32),
        pltpu.VMEM((N_ACC,), jnp.float32),
        pltpu.VMEM((16 * RED,), jnp.float32),
        pltpu.SemaphoreType.DMA,
    ],
    compiler_params=pltpu.CompilerParams(needs_layout_passes=False),
)
def _sc_degree(dst_hbm, zeros_hbm, deg_out, hist_sh, idxv, hist, vbuf, rsem):
    cid = lax.axis_index("c")
    sid = lax.axis_index("s")
    wid = sid * NC + cid

    pltpu.sync_copy(zeros_hbm, hist)
    pltpu.sync_copy(dst_hbm.at[pl.ds(pl.multiple_of(wid * EPW, 8), EPW)], idxv)

    def step(j, carry):
        raw = idxv[pl.ds(j * 16, 16)]
        c_fwd, _ = plsc.scan_count(raw)
        cr, _ = plsc.scan_count(lax.rev(raw, (0,)))
        c_rev = lax.rev(cr, (0,))
        plsc.addupdate_scatter(hist, [raw], c_fwd.astype(jnp.float32),
                               mask=(c_rev == 1))
        return carry

    lax.fori_loop(0, VSTEPS, step, 0)

    pltpu.sync_copy(hist, hist_sh.at[pl.ds(sid * N_ACC, N_ACC)])
    plsc.subcore_barrier()

    def reduce_and_write(red, r0):
        for h in range(NS):
            pltpu.async_copy(hist_sh.at[pl.ds(h * N_ACC + r0, red)],
                             vbuf.at[pl.ds(h * red, red)], rsem)
        for h in range(NS):
            pltpu.make_async_copy(hist_sh.at[pl.ds(h * N_ACC + r0, red)],
                                  vbuf.at[pl.ds(h * red, red)], rsem).wait()

        def vsum(v, carry):
            t = vbuf[pl.ds(v * 16, 16)]
            for h in range(1, NS):
                t = t + vbuf[pl.ds(h * red + v * 16, 16)]
            hist[pl.ds(v * 16, 16)] = t
            return carry

        lax.fori_loop(0, red // 16, vsum, 0)
        pltpu.sync_copy(hist.at[pl.ds(0, red)],
                        deg_out.at[pl.ds(cid * N_ACC + r0, red)])

    @pl.when(sid < NS - 1)
    def _():
        reduce_and_write(RED, sid * RED)

    @pl.when(sid == NS - 1)
    def _():
        reduce_and_write(RED_LAST, (NS - 1) * RED)


@functools.partial(
    pl.kernel,
    out_type=jax.ShapeDtypeStruct((NC, N, D), jnp.float32),
    mesh=_MESH,
    scratch_types=(
        [pltpu.VMEM_SHARED((N_FEAT, D), jnp.float32)]
        + [pltpu.VMEM((K,), jnp.int32)] * 8
        + [pltpu.VMEM((KT,), jnp.int32)] * 2
        + [pltpu.VMEM((K, D), jnp.float32)] * 4
        + [pltpu.SemaphoreType.DMA] * 16
    ),
)
def _sc_scatter(x_hbm, src_hbm, dst_hbm, zeros_hbm, part_out, acc,
                sidx0, sidx1, sidx2, sidx3, didx0, didx1, didx2, didx3,
                sidx_t, didx_t, rows0, rows1, rows2, rows3,
                isem0, isem1, isem2, isem3, gsem0, gsem1, gsem2, gsem3,
                ssem0, ssem1, ssem2, ssem3, dsem0, dsem1, dsem2, dsem3):
    cid = lax.axis_index("c")
    sid = lax.axis_index("s")
    wid = sid * NC + cid

    @pl.when(sid < NS - 1)
    def _():
        pltpu.sync_copy(zeros_hbm, acc.at[pl.ds(sid * ZR, ZR)])

    @pl.when(sid == NS - 1)
    def _():
        pltpu.sync_copy(zeros_hbm.at[pl.ds(0, ZR_LAST)],
                        acc.at[pl.ds((NS - 1) * ZR, ZR_LAST)])

    plsc.subcore_barrier()

    base0 = wid * EPW
    bufs = ((sidx0, didx0, rows0, isem0, gsem0, ssem0, dsem0),
            (sidx1, didx1, rows1, isem1, gsem1, ssem1, dsem1),
            (sidx2, didx2, rows2, isem2, gsem2, ssem2, dsem2),
            (sidx3, didx3, rows3, isem3, gsem3, ssem3, dsem3))
    NB = 4

    def src_slice(k):
        return src_hbm.at[pl.ds(pl.multiple_of(base0 + k * K, 8), K)]

    def dst_slice(k):
        return dst_hbm.at[pl.ds(pl.multiple_of(base0 + k * K, 8), K)]

    def issue_sidx(k, b):
        sidx, _, _, isem, _, _, _ = bufs[b]
        pltpu.async_copy(src_slice(k), sidx, isem)

    def wait_sidx(k, b):
        sidx, _, _, isem, _, _, _ = bufs[b]
        pltpu.make_async_copy(src_slice(k), sidx, isem).wait()

    def issue_didx(k, b):
        _, didx, _, _, _, _, dsem = bufs[b]
        pltpu.async_copy(dst_slice(k), didx, dsem)

    def wait_didx(k, b):
        _, didx, _, _, _, _, dsem = bufs[b]
        pltpu.make_async_copy(dst_slice(k), didx, dsem).wait()

    def issue_gather(b):
        sidx, _, rows, _, gsem, _, _ = bufs[b]
        pltpu.async_copy(x_hbm.at[sidx], rows, gsem)

    def wait_gather(b):
        sidx, _, rows, _, gsem, _, _ = bufs[b]
        pltpu.make_async_copy(x_hbm.at[sidx], rows, gsem).wait()

    def issue_scatter(b):
        _, didx, rows, _, _, ssem, _ = bufs[b]
        pltpu.async_copy(rows, acc.at[didx], ssem, add=True)

    def wait_scatter(b):
        _, didx, rows, _, _, ssem, _ = bufs[b]
        pltpu.make_async_copy(rows, acc.at[didx], ssem).wait()

    for b in range(NB):
        issue_sidx(b, b)
    for b in range(NB):
        issue_didx(b, b)
    for b in range(NB - 1):
        wait_sidx(b, b)
        issue_gather(b)

    def triple(g, carry):
        for b in range(NB):
            k = NB * g + b

            @pl.when(k < CHUNKS)
            def _():
                wait_gather(b)
                nb = (b + NB - 1) % NB

                @pl.when(k + NB < CHUNKS)
                def _():
                    issue_sidx(k + NB, b)

                @pl.when(k + NB - 1 < CHUNKS)
                def _():
                    @pl.when(k >= 1)
                    def _():
                        wait_scatter(nb)
                        issue_didx(k + NB - 1, nb)
                    wait_sidx(k + NB - 1, nb)
                    issue_gather(nb)

                wait_didx(k, b)
                issue_scatter(b)
        return carry

    lax.fori_loop(0, (CHUNKS + NB - 1) // NB, triple, 0)
    for j in range(CHUNKS - NB, CHUNKS):
        wait_scatter(j % NB)
    tb = pl.multiple_of(base0 + CHUNKS * K, 8)
    pltpu.sync_copy(src_hbm.at[pl.ds(tb, KT)], sidx_t)
    pltpu.sync_copy(dst_hbm.at[pl.ds(tb, KT)], didx_t)
    pltpu.async_copy(x_hbm.at[sidx_t], rows0.at[pl.ds(0, KT)], gsem0).wait()
    pltpu.sync_copy(rows0.at[pl.ds(0, KT)], acc.at[didx_t], add=True)
    plsc.subcore_barrier()
    _writeback(sid, cid, acc, part_out)


def _tc_scale_body(x_ref, deg_ref, out_ref):
    d = deg_ref[:, 0:1] + deg_ref[:, 1:2]
    dis = lax.rsqrt(jnp.maximum(d, 1.0))
    out_ref[...] = x_ref[...] * dis


def _tc_scale(x, deg_cols):
    return pl.pallas_call(
        _tc_scale_body,
        grid=(GRID,),
        in_specs=[
            pl.BlockSpec((R_BLK, D), lambda i: (i, 0)),
            pl.BlockSpec((R_BLK, NC), lambda i: (i, 0)),
        ],
        out_specs=pl.BlockSpec((R_BLK, D), lambda i: (i, 0)),
        out_shape=jax.ShapeDtypeStruct((N, D), jnp.float32),
    )(x, deg_cols)


def _tc_final_body(scal_ref, part_ref, deg_ref, h0_ref, w_ref, out_ref):
    theta = scal_ref[0, 0]
    alpha = scal_ref[0, 1]
    d = deg_ref[:, 0:1] + deg_ref[:, 1:2]
    dis = lax.rsqrt(jnp.maximum(d, 1.0))
    h_acc = part_ref[0] + part_ref[1]
    support = (1.0 - alpha) * (h_acc * dis) + alpha * h0_ref[...]
    mm = jnp.dot(support, w_ref[...], preferred_element_type=jnp.float32)
    out_ref[...] = theta * mm + (1.0 - theta) * support


def _tc_final(part, deg_cols, h0, W, scal):
    return pl.pallas_call(
        _tc_final_body,
        grid=(GRID,),
        in_specs=[
            pl.BlockSpec(memory_space=pltpu.SMEM),
            pl.BlockSpec((NC, R_BLK, D), lambda i: (0, i, 0)),
            pl.BlockSpec((R_BLK, NC), lambda i: (i, 0)),
            pl.BlockSpec((R_BLK, D), lambda i: (i, 0)),
            pl.BlockSpec((D, D), lambda i: (0, 0)),
        ],
        out_specs=pl.BlockSpec((R_BLK, D), lambda i: (i, 0)),
        out_shape=jax.ShapeDtypeStruct((N, D), jnp.float32),
    )(scal, part, deg_cols, h0, W)


def kernel(inputs, edge_index, h0, lamda, alpha, l, W):
    theta = jnp.log(lamda / l + 1)
    scal = jnp.reshape(
        jnp.stack([theta, alpha]).astype(jnp.float32), (1, 2))

    src = edge_index[0]
    dst = edge_index[1]

    zstripe = jnp.zeros((ZR, D), jnp.float32)
    zeros1 = jnp.zeros((N_ACC,), jnp.float32)

    deg_flat = _sc_degree(dst, zeros1)
    deg_cols = jnp.transpose(jnp.reshape(deg_flat, (NC, N_ACC)))[:N]
    x_scaled = _tc_scale(inputs, deg_cols)
    part = _sc_scatter(x_scaled, src, dst, zstripe)
    return _tc_final(part, deg_cols, h0, W, scal)


# speedup vs baseline: 1.4524x; 1.0009x over previous
import functools

import jax
import jax.numpy as jnp
from jax import lax
from jax.experimental import pallas as pl
from jax.experimental.pallas import tpu as pltpu
from jax.experimental.pallas import tpu_sc as plsc

N = 10000
E = 320000
D = 128

NC = 2
NS = 16
NW = NC * NS

K = 96
EPW = E // NW
CHUNKS = EPW // K
KT = EPW - CHUNKS * K

N_ACC = 10112
N_FEAT = 10040
ZR = 632
ZR_LAST = N_FEAT - 15 * ZR
OUTR = 632
OUTR_LAST = N - 15 * OUTR

R_BLK = 2000
GRID = N // R_BLK

_MESH = plsc.VectorSubcoreMesh(core_axis_name="c", subcore_axis_name="s")


def _writeback(sid, cid, acc, out_hbm):
    r0 = sid * OUTR

    @pl.when(sid < NS - 1)
    def _():
        pltpu.sync_copy(acc.at[pl.ds(r0, OUTR)],
                        out_hbm.at[cid, pl.ds(r0, OUTR)])

    @pl.when(sid == NS - 1)
    def _():
        r1 = (NS - 1) * OUTR
        pltpu.sync_copy(acc.at[pl.ds(r1, OUTR_LAST)],
                        out_hbm.at[cid, pl.ds(r1, OUTR_LAST)])


RED = 640
RED_LAST = N_ACC - 15 * RED
VSTEPS = EPW // 16


@functools.partial(
    pl.kernel,
    out_type=jax.ShapeDtypeStruct((NC * N_ACC,), jnp.float32),
    mesh=_MESH,
    scratch_types=[
        pltpu.VMEM_SHARED((NS * N_ACC,), jnp.float32),
        pltpu.VMEM((EPW,), jnp.int32),
        pltpu.VMEM((N_ACC,), jnp.float32),
        pltpu.VMEM((16 * RED,), jnp.float32),
        pltpu.SemaphoreType.DMA,
    ],
    compiler_params=pltpu.CompilerParams(needs_layout_passes=False),
)
def _sc_degree(dst_hbm, zeros_hbm, deg_out, hist_sh, idxv, hist, vbuf, rsem):
    cid = lax.axis_index("c")
    sid = lax.axis_index("s")
    wid = sid * NC + cid

    pltpu.sync_copy(zeros_hbm, hist)
    pltpu.sync_copy(dst_hbm.at[pl.ds(pl.multiple_of(wid * EPW, 8), EPW)], idxv)

    def step(j, carry):
        for u in range(5):
            raw = idxv[pl.ds((5 * j + u) * 16, 16)]
            c_fwd, _ = plsc.scan_count(raw)
            cr, _ = plsc.scan_count(lax.rev(raw, (0,)))
            c_rev = lax.rev(cr, (0,))
            plsc.addupdate_scatter(hist, [raw], c_fwd.astype(jnp.float32),
                                   mask=(c_rev == 1))
        return carry

    lax.fori_loop(0, VSTEPS // 5, step, 0)

    pltpu.sync_copy(hist, hist_sh.at[pl.ds(sid * N_ACC, N_ACC)])
    plsc.subcore_barrier()

    def reduce_and_write(red, r0):
        for h in range(NS):
            pltpu.async_copy(hist_sh.at[pl.ds(h * N_ACC + r0, red)],
                             vbuf.at[pl.ds(h * red, red)], rsem)
        for h in range(NS):
            pltpu.make_async_copy(hist_sh.at[pl.ds(h * N_ACC + r0, red)],
                                  vbuf.at[pl.ds(h * red, red)], rsem).wait()

        def vsum(v, carry):
            t = vbuf[pl.ds(v * 16, 16)]
            for h in range(1, NS):
                t = t + vbuf[pl.ds(h * red + v * 16, 16)]
            hist[pl.ds(v * 16, 16)] = t
            return carry

        lax.fori_loop(0, red // 16, vsum, 0)
        pltpu.sync_copy(hist.at[pl.ds(0, red)],
                        deg_out.at[pl.ds(cid * N_ACC + r0, red)])

    @pl.when(sid < NS - 1)
    def _():
        reduce_and_write(RED, sid * RED)

    @pl.when(sid == NS - 1)
    def _():
        reduce_and_write(RED_LAST, (NS - 1) * RED)


@functools.partial(
    pl.kernel,
    out_type=jax.ShapeDtypeStruct((NC, N, D), jnp.float32),
    mesh=_MESH,
    scratch_types=(
        [pltpu.VMEM_SHARED((N_FEAT, D), jnp.float32)]
        + [pltpu.VMEM((K,), jnp.int32)] * 8
        + [pltpu.VMEM((KT,), jnp.int32)] * 2
        + [pltpu.VMEM((K, D), jnp.float32)] * 4
        + [pltpu.SemaphoreType.DMA] * 16
    ),
)
def _sc_scatter(x_hbm, src_hbm, dst_hbm, zeros_hbm, part_out, acc,
                sidx0, sidx1, sidx2, sidx3, didx0, didx1, didx2, didx3,
                sidx_t, didx_t, rows0, rows1, rows2, rows3,
                isem0, isem1, isem2, isem3, gsem0, gsem1, gsem2, gsem3,
                ssem0, ssem1, ssem2, ssem3, dsem0, dsem1, dsem2, dsem3):
    cid = lax.axis_index("c")
    sid = lax.axis_index("s")
    wid = sid * NC + cid

    @pl.when(sid < NS - 1)
    def _():
        pltpu.sync_copy(zeros_hbm, acc.at[pl.ds(sid * ZR, ZR)])

    @pl.when(sid == NS - 1)
    def _():
        pltpu.sync_copy(zeros_hbm.at[pl.ds(0, ZR_LAST)],
                        acc.at[pl.ds((NS - 1) * ZR, ZR_LAST)])

    plsc.subcore_barrier()

    base0 = wid * EPW
    bufs = ((sidx0, didx0, rows0, isem0, gsem0, ssem0, dsem0),
            (sidx1, didx1, rows1, isem1, gsem1, ssem1, dsem1),
            (sidx2, didx2, rows2, isem2, gsem2, ssem2, dsem2),
            (sidx3, didx3, rows3, isem3, gsem3, ssem3, dsem3))
    NB = 4

    def src_slice(k):
        return src_hbm.at[pl.ds(pl.multiple_of(base0 + k * K, 8), K)]

    def dst_slice(k):
        return dst_hbm.at[pl.ds(pl.multiple_of(base0 + k * K, 8), K)]

    def issue_sidx(k, b):
        sidx, _, _, isem, _, _, _ = bufs[b]
        pltpu.async_copy(src_slice(k), sidx, isem)

    def wait_sidx(k, b):
        sidx, _, _, isem, _, _, _ = bufs[b]
        pltpu.make_async_copy(src_slice(k), sidx, isem).wait()

    def issue_didx(k, b):
        _, didx, _, _, _, _, dsem = bufs[b]
        pltpu.async_copy(dst_slice(k), didx, dsem)

    def wait_didx(k, b):
        _, didx, _, _, _, _, dsem = bufs[b]
        pltpu.make_async_copy(dst_slice(k), didx, dsem).wait()

    def issue_gather(b):
        sidx, _, rows, _, gsem, _, _ = bufs[b]
        pltpu.async_copy(x_hbm.at[sidx], rows, gsem)

    def wait_gather(b):
        sidx, _, rows, _, gsem, _, _ = bufs[b]
        pltpu.make_async_copy(x_hbm.at[sidx], rows, gsem).wait()

    def issue_scatter(b):
        _, didx, rows, _, _, ssem, _ = bufs[b]
        pltpu.async_copy(rows, acc.at[didx], ssem, add=True)

    def wait_scatter(b):
        _, didx, rows, _, _, ssem, _ = bufs[b]
        pltpu.make_async_copy(rows, acc.at[didx], ssem).wait()

    for b in range(NB):
        issue_sidx(b, b)
    for b in range(NB):
        issue_didx(b, b)
    for b in range(NB - 1):
        wait_sidx(b, b)
        issue_gather(b)

    def triple(g, carry):
        for b in range(NB):
            k = NB * g + b

            @pl.when(k < CHUNKS)
            def _():
                wait_gather(b)
                nb = (b + NB - 1) % NB

                @pl.when(k + NB < CHUNKS)
                def _():
                    issue_sidx(k + NB, b)

                @pl.when(k + NB - 1 < CHUNKS)
                def _():
                    @pl.when(k >= 1)
                    def _():
                        wait_scatter(nb)
                        issue_didx(k + NB - 1, nb)
                    wait_sidx(k + NB - 1, nb)
                    issue_gather(nb)

                wait_didx(k, b)
                issue_scatter(b)
        return carry

    lax.fori_loop(0, (CHUNKS + NB - 1) // NB, triple, 0)
    for j in range(CHUNKS - NB, CHUNKS):
        wait_scatter(j % NB)
    tb = pl.multiple_of(base0 + CHUNKS * K, 8)
    pltpu.sync_copy(src_hbm.at[pl.ds(tb, KT)], sidx_t)
    pltpu.sync_copy(dst_hbm.at[pl.ds(tb, KT)], didx_t)
    pltpu.async_copy(x_hbm.at[sidx_t], rows0.at[pl.ds(0, KT)], gsem0).wait()
    pltpu.sync_copy(rows0.at[pl.ds(0, KT)], acc.at[didx_t], add=True)
    plsc.subcore_barrier()
    _writeback(sid, cid, acc, part_out)


def _tc_scale_body(x_ref, deg_ref, out_ref):
    d = deg_ref[:, 0:1] + deg_ref[:, 1:2]
    dis = lax.rsqrt(jnp.maximum(d, 1.0))
    out_ref[...] = x_ref[...] * dis


def _tc_scale(x, deg_cols):
    return pl.pallas_call(
        _tc_scale_body,
        grid=(GRID,),
        in_specs=[
            pl.BlockSpec((R_BLK, D), lambda i: (i, 0)),
            pl.BlockSpec((R_BLK, NC), lambda i: (i, 0)),
        ],
        out_specs=pl.BlockSpec((R_BLK, D), lambda i: (i, 0)),
        out_shape=jax.ShapeDtypeStruct((N, D), jnp.float32),
    )(x, deg_cols)


def _tc_final_body(scal_ref, part_ref, deg_ref, h0_ref, w_ref, out_ref):
    theta = scal_ref[0, 0]
    alpha = scal_ref[0, 1]
    d = deg_ref[:, 0:1] + deg_ref[:, 1:2]
    dis = lax.rsqrt(jnp.maximum(d, 1.0))
    h_acc = part_ref[0] + part_ref[1]
    support = (1.0 - alpha) * (h_acc * dis) + alpha * h0_ref[...]
    mm = jnp.dot(support, w_ref[...], preferred_element_type=jnp.float32)
    out_ref[...] = theta * mm + (1.0 - theta) * support


def _tc_final(part, deg_cols, h0, W, scal):
    return pl.pallas_call(
        _tc_final_body,
        grid=(GRID,),
        in_specs=[
            pl.BlockSpec(memory_space=pltpu.SMEM),
            pl.BlockSpec((NC, R_BLK, D), lambda i: (0, i, 0)),
            pl.BlockSpec((R_BLK, NC), lambda i: (i, 0)),
            pl.BlockSpec((R_BLK, D), lambda i: (i, 0)),
            pl.BlockSpec((D, D), lambda i: (0, 0)),
        ],
        out_specs=pl.BlockSpec((R_BLK, D), lambda i: (i, 0)),
        out_shape=jax.ShapeDtypeStruct((N, D), jnp.float32),
    )(scal, part, deg_cols, h0, W)


def kernel(inputs, edge_index, h0, lamda, alpha, l, W):
    theta = jnp.log(lamda / l + 1)
    scal = jnp.reshape(
        jnp.stack([theta, alpha]).astype(jnp.float32), (1, 2))

    src = edge_index[0]
    dst = edge_index[1]

    zstripe = jnp.zeros((ZR, D), jnp.float32)
    zeros1 = jnp.zeros((N_ACC,), jnp.float32)

    deg_flat = _sc_degree(dst, zeros1)
    deg_cols = jnp.transpose(jnp.reshape(deg_flat, (NC, N_ACC)))[:N]
    x_scaled = _tc_scale(inputs, deg_cols)
    part = _sc_scatter(x_scaled, src, dst, zstripe)
    return _tc_final(part, deg_cols, h0, W, scal)
